# baseline (speedup 1.0000x reference)
import functools

import jax
import jax.numpy as jnp
from jax import lax
from jax.experimental import pallas as pl
from jax.experimental.pallas import tpu as pltpu
from jax.experimental.pallas import tpu_sc as plsc

N = 10000
E = 320000
D = 128
DE = 16

NC = 2
NS = 16
NW = NC * NS
CH = 128
NJ = 79
E_PAD = NW * NJ * CH
PADC = E_PAD - E
NFULL = N // CH
NTAIL = N - NFULL * CH


def _ceil_div(a, b):
    return (a + b - 1) // b


def _make_segsum(gather: bool):
    mesh = plsc.VectorSubcoreMesh(core_axis_name="c", subcore_axis_name="s",
                                  num_cores=NC, num_subcores=NS)

    @functools.partial(
        pl.kernel,
        out_type=jax.ShapeDtypeStruct((NC, N, D), jnp.float32),
        mesh=mesh,
        scratch_types=[
            pltpu.VMEM((NJ, CH), jnp.int32),
            pltpu.VMEM((NJ, CH), jnp.int32),
            pltpu.VMEM((CH,), jnp.int32),
            pltpu.VMEM((CH,), jnp.int32),
            pltpu.VMEM((CH, D), jnp.float32),
            pltpu.VMEM_SHARED((N, D), jnp.float32),
            pltpu.SemaphoreType.DMA,
        ],
    )
    def segsum(vals_hbm, sidx_hbm, didx_hbm, zeros_hbm, out_hbm,
               sidx, didx, didxc, sidxc, rows, acc, gsem):
        c = lax.axis_index("c")
        s = lax.axis_index("s")
        w = c * NS + s

        for t in range(_ceil_div(NFULL, NS)):
            j = s + NS * t
            @pl.when(j < NFULL)
            def _():
                r0 = pl.multiple_of(j * CH, CH)
                pltpu.sync_copy(zeros_hbm, acc.at[pl.ds(r0, CH)])
        @pl.when(s == 0)
        def _():
            pltpu.sync_copy(zeros_hbm.at[pl.ds(0, NTAIL)],
                            acc.at[pl.ds(NFULL * CH, NTAIL)])

        if gather:
            pltpu.sync_copy(sidx_hbm.at[w], sidx)
        pltpu.sync_copy(didx_hbm.at[w], didx)
        plsc.subcore_barrier()

        base0 = w * NJ * CH

        def body(j):
            if gather:
                for k in range(CH // 16):
                    sidxc[pl.ds(16 * k, 16)] = sidx[j, pl.ds(16 * k, 16)]
                d = pltpu.async_copy(vals_hbm.at[sidxc], rows, gsem)
            else:
                d = pltpu.async_copy(
                    vals_hbm.at[pl.ds(base0 + j * CH, CH)], rows, gsem)
            for k in range(CH // 16):
                didxc[pl.ds(16 * k, 16)] = didx[j, pl.ds(16 * k, 16)]
            d.wait()
            pltpu.sync_copy(rows, acc.at[didxc], add=True)

        def trip(j, carry):
            if gather:
                body(j)
            else:
                @pl.when(base0 + j * CH < E)
                def _():
                    body(j)
            return carry

        lax.fori_loop(0, NJ, trip, 0)
        plsc.subcore_barrier()

        for t in range(_ceil_div(NFULL, NS)):
            j = s + NS * t
            @pl.when(j < NFULL)
            def _():
                r0 = pl.multiple_of(j * CH, CH)
                pltpu.sync_copy(acc.at[pl.ds(r0, CH)], rows)
                pltpu.sync_copy(rows, out_hbm.at[c, pl.ds(r0, CH)])
        @pl.when(s == 0)
        def _():
            r0 = NFULL * CH
            pltpu.sync_copy(acc.at[pl.ds(r0, NTAIL)], rows.at[pl.ds(0, NTAIL)])
            pltpu.sync_copy(rows.at[pl.ds(0, NTAIL)],
                            out_hbm.at[c, pl.ds(r0, NTAIL)])

    return segsum


_segsum_linear = _make_segsum(False)
_segsum_gather = _make_segsum(True)


_BE = 8000
_BR = 2000


def _edge_linear_body(x_ref, w_ref, o_ref):
    o_ref[...] = jax.nn.relu(
        jnp.dot(x_ref[...], w_ref[...], preferred_element_type=jnp.float32))


def _edge_linear(edge_feat, W_e2l):
    return pl.pallas_call(
        _edge_linear_body,
        grid=(E // _BE,),
        in_specs=[
            pl.BlockSpec((_BE, DE), lambda i: (i, 0)),
            pl.BlockSpec((DE, D), lambda i: (0, 0)),
        ],
        out_specs=pl.BlockSpec((_BE, D), lambda i: (i, 0)),
        out_shape=jax.ShapeDtypeStruct((E, D), jnp.float32),
    )(edge_feat, W_e2l)


def _combine0_body(p_ref, w_ref, static_ref, cur_ref):
    pool = p_ref[0] + p_ref[1]
    sm = jnp.dot(pool, w_ref[...], preferred_element_type=jnp.float32)
    static_ref[...] = sm
    cur_ref[...] = jax.nn.relu(sm)


def _combine0(p, W0):
    return pl.pallas_call(
        _combine0_body,
        grid=(N // _BR,),
        in_specs=[
            pl.BlockSpec((NC, _BR, D), lambda i: (0, i, 0)),
            pl.BlockSpec((D, D), lambda i: (0, 0)),
        ],
        out_specs=[
            pl.BlockSpec((_BR, D), lambda i: (i, 0)),
            pl.BlockSpec((_BR, D), lambda i: (i, 0)),
        ],
        out_shape=[
            jax.ShapeDtypeStruct((N, D), jnp.float32),
            jax.ShapeDtypeStruct((N, D), jnp.float32),
        ],
    )(p, W0)


def _level_body(cur_ref, t0_ref, p_ref, static_ref, w1_ref, w2_ref, o_ref):
    cur = cur_ref[...]
    glob = pl.program_id(0) * _BR + lax.broadcasted_iota(jnp.int32, (_BR, 1), 0)
    pool = (p_ref[0] + p_ref[1]
            - jnp.where(glob < PADC, 1.0, 0.0) * t0_ref[0:1])
    acc = static_ref[...]
    acc += jnp.dot(cur, w1_ref[...], preferred_element_type=jnp.float32)
    acc += jnp.dot(pool, w2_ref[...], preferred_element_type=jnp.float32)
    o_ref[...] = jax.nn.relu(acc)


def _level(cur, p, static, W1, W2):
    return pl.pallas_call(
        _level_body,
        grid=(N // _BR,),
        in_specs=[
            pl.BlockSpec((_BR, D), lambda i: (i, 0)),
            pl.BlockSpec((8, D), lambda i: (0, 0)),
            pl.BlockSpec((NC, _BR, D), lambda i: (0, i, 0)),
            pl.BlockSpec((_BR, D), lambda i: (i, 0)),
            pl.BlockSpec((D, D), lambda i: (0, 0)),
            pl.BlockSpec((D, D), lambda i: (0, 0)),
        ],
        out_specs=pl.BlockSpec((_BR, D), lambda i: (i, 0)),
        out_shape=jax.ShapeDtypeStruct((N, D), jnp.float32),
    )(cur, cur, p, static, W1, W2)


def _pad_concat(x, pad_vals):
    return jnp.concatenate([x.astype(jnp.int32), pad_vals]).reshape(NW, NJ, CH)


def kernel(edge_feat, edge_index, W_e2l, W0, W11, W21, W12, W22, W13, W23):
    src = _pad_concat(edge_index[0], jnp.zeros((PADC,), jnp.int32))
    dst = _pad_concat(edge_index[1], jnp.arange(PADC, dtype=jnp.int32))
    zeros = jnp.zeros((CH, D), jnp.float32)

    edge_lin = _edge_linear(edge_feat, W_e2l)
    p = _segsum_linear(edge_lin, src, dst, zeros)
    static, cur = _combine0(p, W0)
    for W1, W2 in ((W11, W21), (W12, W22), (W13, W23)):
        p = _segsum_gather(cur, src, dst, zeros)
        cur = _level(cur, p, static, W1, W2)
    return cur


# speedup vs baseline: 1.5012x; 1.5012x over previous
import functools

import jax
import jax.numpy as jnp
from jax import lax
from jax.experimental import pallas as pl
from jax.experimental.pallas import tpu as pltpu
from jax.experimental.pallas import tpu_sc as plsc

N = 10000
E = 320000
D = 128
DE = 16

NC = 2
NS = 16
CH = 128
NCHUNK = E // CH
CPC = NCHUNK // NC
TRIPS = (CPC + NS - 1) // NS
NFULL = N // CH
NTAIL = N - NFULL * CH


def _ceil_div(a, b):
    return (a + b - 1) // b


def _make_segsum(gather: bool):
    mesh = plsc.VectorSubcoreMesh(core_axis_name="c", subcore_axis_name="s",
                                  num_cores=NC, num_subcores=NS)

    @functools.partial(
        pl.kernel,
        out_type=jax.ShapeDtypeStruct((NC, N, D), jnp.float32),
        mesh=mesh,
        scratch_types=[
            pltpu.VMEM((CH,), jnp.int32),
            pltpu.VMEM((CH,), jnp.int32),
            pltpu.VMEM((CH,), jnp.int32),
            pltpu.VMEM((CH,), jnp.int32),
            pltpu.VMEM((CH, D), jnp.float32),
            pltpu.VMEM_SHARED((N, D), jnp.float32),
            pltpu.SemaphoreType.DMA,
            pltpu.SemaphoreType.DMA,
            pltpu.SemaphoreType.DMA,
        ],
    )
    def segsum(vals_hbm, src_hbm, dst_hbm, zeros_hbm, out_hbm,
               sidx0, sidx1, didx0, didx1, rows, acc, gsem, isem0, isem1):
        c = lax.axis_index("c")
        s = lax.axis_index("s")
        sidx = (sidx0, sidx1)
        didx = (didx0, didx1)
        isem = (isem0, isem1)

        for t in range(_ceil_div(NFULL, NS)):
            j = s + NS * t
            @pl.when(j < NFULL)
            def _():
                r0 = pl.multiple_of(j * CH, CH)
                pltpu.sync_copy(zeros_hbm, acc.at[pl.ds(r0, CH)])
        @pl.when(s == 0)
        def _():
            pltpu.sync_copy(zeros_hbm.at[pl.ds(0, NTAIL)],
                            acc.at[pl.ds(NFULL * CH, NTAIL)])
        plsc.subcore_barrier()

        def local(t):
            return t * NS + s

        def base(t):
            return pl.multiple_of((c * CPC + local(t)) * CH, CH)

        def idx_fetch(t, bank):
            @pl.when(local(t) < CPC)
            def _():
                if gather:
                    pltpu.make_async_copy(
                        src_hbm.at[pl.ds(base(t), CH)], sidx[bank],
                        isem[bank]).start()
                pltpu.make_async_copy(
                    dst_hbm.at[pl.ds(base(t), CH)], didx[bank],
                    isem[bank]).start()

        def idx_wait(t, bank):
            if gather:
                pltpu.make_async_copy(
                    src_hbm.at[pl.ds(base(t), CH)], sidx[bank],
                    isem[bank]).wait()
            pltpu.make_async_copy(
                dst_hbm.at[pl.ds(base(t), CH)], didx[bank],
                isem[bank]).wait()

        idx_fetch(0, 0)

        def trip(i, carry):
            for b in range(2):
                t = i * 2 + b
                @pl.when(local(t) < CPC)
                def _():
                    idx_wait(t, b)
                    @pl.when(local(t + 1) < CPC)
                    def _():
                        idx_fetch(t + 1, 1 - b)
                    if gather:
                        pltpu.async_copy(
                            vals_hbm.at[sidx[b]], rows, gsem).wait()
                    else:
                        pltpu.async_copy(
                            vals_hbm.at[pl.ds(base(t), CH)], rows, gsem).wait()
                    pltpu.sync_copy(rows, acc.at[didx[b]], add=True)
            return carry

        lax.fori_loop(0, _ceil_div(TRIPS, 2), trip, 0)
        plsc.subcore_barrier()

        for t in range(_ceil_div(NFULL, NS)):
            j = s + NS * t
            @pl.when(j < NFULL)
            def _():
                r0 = pl.multiple_of(j * CH, CH)
                pltpu.sync_copy(acc.at[pl.ds(r0, CH)], rows)
                pltpu.sync_copy(rows, out_hbm.at[c, pl.ds(r0, CH)])
        @pl.when(s == 0)
        def _():
            r0 = NFULL * CH
            pltpu.sync_copy(acc.at[pl.ds(r0, NTAIL)], rows.at[pl.ds(0, NTAIL)])
            pltpu.sync_copy(rows.at[pl.ds(0, NTAIL)],
                            out_hbm.at[c, pl.ds(r0, NTAIL)])

    return segsum


_segsum_linear = _make_segsum(False)
_segsum_gather = _make_segsum(True)


_BE = 8000
_BR = 2000


def _edge_linear_body(x_ref, w_ref, o_ref):
    o_ref[...] = jax.nn.relu(
        jnp.dot(x_ref[...], w_ref[...], preferred_element_type=jnp.float32))


def _edge_linear(edge_feat, W_e2l):
    return pl.pallas_call(
        _edge_linear_body,
        grid=(E // _BE,),
        in_specs=[
            pl.BlockSpec((_BE, DE), lambda i: (i, 0)),
            pl.BlockSpec((DE, D), lambda i: (0, 0)),
        ],
        out_specs=pl.BlockSpec((_BE, D), lambda i: (i, 0)),
        out_shape=jax.ShapeDtypeStruct((E, D), jnp.float32),
    )(edge_feat, W_e2l)


def _combine0_body(p_ref, w_ref, static_ref, cur_ref):
    pool = p_ref[0] + p_ref[1]
    sm = jnp.dot(pool, w_ref[...], preferred_element_type=jnp.float32)
    static_ref[...] = sm
    cur_ref[...] = jax.nn.relu(sm)


def _combine0(p, W0):
    return pl.pallas_call(
        _combine0_body,
        grid=(N // _BR,),
        in_specs=[
            pl.BlockSpec((NC, _BR, D), lambda i: (0, i, 0)),
            pl.BlockSpec((D, D), lambda i: (0, 0)),
        ],
        out_specs=[
            pl.BlockSpec((_BR, D), lambda i: (i, 0)),
            pl.BlockSpec((_BR, D), lambda i: (i, 0)),
        ],
        out_shape=[
            jax.ShapeDtypeStruct((N, D), jnp.float32),
            jax.ShapeDtypeStruct((N, D), jnp.float32),
        ],
    )(p, W0)


def _level_body(cur_ref, p_ref, static_ref, w1_ref, w2_ref, o_ref):
    pool = p_ref[0] + p_ref[1]
    acc = static_ref[...]
    acc += jnp.dot(cur_ref[...], w1_ref[...], preferred_element_type=jnp.float32)
    acc += jnp.dot(pool, w2_ref[...], preferred_element_type=jnp.float32)
    o_ref[...] = jax.nn.relu(acc)


def _level(cur, p, static, W1, W2):
    return pl.pallas_call(
        _level_body,
        grid=(N // _BR,),
        in_specs=[
            pl.BlockSpec((_BR, D), lambda i: (i, 0)),
            pl.BlockSpec((NC, _BR, D), lambda i: (0, i, 0)),
            pl.BlockSpec((_BR, D), lambda i: (i, 0)),
            pl.BlockSpec((D, D), lambda i: (0, 0)),
            pl.BlockSpec((D, D), lambda i: (0, 0)),
        ],
        out_specs=pl.BlockSpec((_BR, D), lambda i: (i, 0)),
        out_shape=jax.ShapeDtypeStruct((N, D), jnp.float32),
    )(cur, p, static, W1, W2)


def kernel(edge_feat, edge_index, W_e2l, W0, W11, W21, W12, W22, W13, W23):
    src = edge_index[0]
    dst = edge_index[1]
    zeros = jnp.zeros((CH, D), jnp.float32)

    edge_lin = _edge_linear(edge_feat, W_e2l)
    p = _segsum_linear(edge_lin, src, dst, zeros)
    static, cur = _combine0(p, W0)
    for W1, W2 in ((W11, W21), (W12, W22), (W13, W23)):
        p = _segsum_gather(cur, src, dst, zeros)
        cur = _level(cur, p, static, W1, W2)
    return cur


# speedup vs baseline: 1.8550x; 1.2357x over previous
import functools

import jax
import jax.numpy as jnp
from jax import lax
from jax.experimental import pallas as pl
from jax.experimental.pallas import tpu as pltpu
from jax.experimental.pallas import tpu_sc as plsc

N = 10000
E = 320000
D = 128
DE = 16

NC = 2
NS = 16
CH = 128
NCHUNK = E // CH
CPC = NCHUNK // NC
TRIPS = (CPC + NS - 1) // NS
NFULL = N // CH
NTAIL = N - NFULL * CH


def _ceil_div(a, b):
    return (a + b - 1) // b


def _make_segsum(gather: bool):
    mesh = plsc.VectorSubcoreMesh(core_axis_name="c", subcore_axis_name="s",
                                  num_cores=NC, num_subcores=NS)

    @functools.partial(
        pl.kernel,
        out_type=jax.ShapeDtypeStruct((NC, N, D), jnp.float32),
        mesh=mesh,
        scratch_types=[
            pltpu.VMEM((CH,), jnp.int32),
            pltpu.VMEM((CH,), jnp.int32),
            pltpu.VMEM((CH,), jnp.int32),
            pltpu.VMEM((CH,), jnp.int32),
            pltpu.VMEM((CH,), jnp.int32),
            pltpu.VMEM((CH,), jnp.int32),
            pltpu.VMEM((CH, D), jnp.float32),
            pltpu.VMEM((CH, D), jnp.float32),
            pltpu.VMEM_SHARED((N, D), jnp.float32),
            pltpu.SemaphoreType.DMA,
            pltpu.SemaphoreType.DMA,
            pltpu.SemaphoreType.DMA,
            pltpu.SemaphoreType.DMA,
            pltpu.SemaphoreType.DMA,
        ],
    )
    def segsum(vals_hbm, src_hbm, dst_hbm, zeros_hbm, out_hbm,
               sidx0, sidx1, sidx2, didx0, didx1, didx2, rows0, rows1,
               acc, g0, g1, i0, i1, i2):
        c = lax.axis_index("c")
        s = lax.axis_index("s")
        sidx = (sidx0, sidx1, sidx2)
        didx = (didx0, didx1, didx2)
        isem = (i0, i1, i2)
        rows = (rows0, rows1)
        gsem = (g0, g1)

        for t in range(_ceil_div(NFULL, NS)):
            j = s + NS * t
            @pl.when(j < NFULL)
            def _():
                r0 = pl.multiple_of(j * CH, CH)
                pltpu.sync_copy(zeros_hbm, acc.at[pl.ds(r0, CH)])
        @pl.when(s == 0)
        def _():
            pltpu.sync_copy(zeros_hbm.at[pl.ds(0, NTAIL)],
                            acc.at[pl.ds(NFULL * CH, NTAIL)])
        plsc.subcore_barrier()

        def local(t):
            return t * NS + s

        def base(t):
            return pl.multiple_of((c * CPC + local(t)) * CH, CH)

        def idx_fetch(t, k):
            @pl.when(local(t) < CPC)
            def _():
                if gather:
                    pltpu.make_async_copy(
                        src_hbm.at[pl.ds(base(t), CH)], sidx[k],
                        isem[k]).start()
                pltpu.make_async_copy(
                    dst_hbm.at[pl.ds(base(t), CH)], didx[k],
                    isem[k]).start()

        def idx_wait(t, k):
            if gather:
                pltpu.make_async_copy(
                    src_hbm.at[pl.ds(base(t), CH)], sidx[k],
                    isem[k]).wait()
            pltpu.make_async_copy(
                dst_hbm.at[pl.ds(base(t), CH)], didx[k],
                isem[k]).wait()

        def gather_copy(t, k, b):
            if gather:
                return pltpu.make_async_copy(
                    vals_hbm.at[sidx[k]], rows[b], gsem[b])
            return pltpu.make_async_copy(
                vals_hbm.at[pl.ds(base(t), CH)], rows[b], gsem[b])

        idx_fetch(0, 0)
        idx_fetch(1, 1)
        @pl.when(local(0) < CPC)
        def _():
            idx_wait(0, 0)
            gather_copy(0, 0, 0).start()

        def trip(i, carry):
            for u in range(6):
                t = i * 6 + u
                b = u % 2
                k = u % 3
                k1 = (u + 1) % 3
                k2 = (u + 2) % 3
                @pl.when(local(t) < CPC)
                def _():
                    gather_copy(t, k, b).wait()
                    @pl.when(local(t + 1) < CPC)
                    def _():
                        idx_wait(t + 1, k1)
                        gather_copy(t + 1, k1, 1 - b).start()
                    @pl.when(local(t + 2) < CPC)
                    def _():
                        idx_fetch(t + 2, k2)
                    pltpu.sync_copy(rows[b], acc.at[didx[k]], add=True)
            return carry

        lax.fori_loop(0, _ceil_div(TRIPS, 6), trip, 0)
        plsc.subcore_barrier()

        for t in range(_ceil_div(NFULL, NS)):
            j = s + NS * t
            @pl.when(j < NFULL)
            def _():
                r0 = pl.multiple_of(j * CH, CH)
                pltpu.sync_copy(acc.at[pl.ds(r0, CH)], rows0)
                pltpu.sync_copy(rows0, out_hbm.at[c, pl.ds(r0, CH)])
        @pl.when(s == 0)
        def _():
            r0 = NFULL * CH
            pltpu.sync_copy(acc.at[pl.ds(r0, NTAIL)], rows1.at[pl.ds(0, NTAIL)])
            pltpu.sync_copy(rows1.at[pl.ds(0, NTAIL)],
                            out_hbm.at[c, pl.ds(r0, NTAIL)])

    return segsum


_segsum_linear = _make_segsum(False)
_segsum_gather = _make_segsum(True)


_BE = 8000
_BR = 2000


def _edge_linear_body(x_ref, w_ref, o_ref):
    o_ref[...] = jax.nn.relu(
        jnp.dot(x_ref[...], w_ref[...], preferred_element_type=jnp.float32))


def _edge_linear(edge_feat, W_e2l):
    return pl.pallas_call(
        _edge_linear_body,
        grid=(E // _BE,),
        in_specs=[
            pl.BlockSpec((_BE, DE), lambda i: (i, 0)),
            pl.BlockSpec((DE, D), lambda i: (0, 0)),
        ],
        out_specs=pl.BlockSpec((_BE, D), lambda i: (i, 0)),
        out_shape=jax.ShapeDtypeStruct((E, D), jnp.float32),
    )(edge_feat, W_e2l)


def _combine0_body(p_ref, w_ref, static_ref, cur_ref):
    pool = p_ref[0] + p_ref[1]
    sm = jnp.dot(pool, w_ref[...], preferred_element_type=jnp.float32)
    static_ref[...] = sm
    cur_ref[...] = jax.nn.relu(sm)


def _combine0(p, W0):
    return pl.pallas_call(
        _combine0_body,
        grid=(N // _BR,),
        in_specs=[
            pl.BlockSpec((NC, _BR, D), lambda i: (0, i, 0)),
            pl.BlockSpec((D, D), lambda i: (0, 0)),
        ],
        out_specs=[
            pl.BlockSpec((_BR, D), lambda i: (i, 0)),
            pl.BlockSpec((_BR, D), lambda i: (i, 0)),
        ],
        out_shape=[
            jax.ShapeDtypeStruct((N, D), jnp.float32),
            jax.ShapeDtypeStruct((N, D), jnp.float32),
        ],
    )(p, W0)


def _level_body(cur_ref, p_ref, static_ref, w1_ref, w2_ref, o_ref):
    pool = p_ref[0] + p_ref[1]
    acc = static_ref[...]
    acc += jnp.dot(cur_ref[...], w1_ref[...], preferred_element_type=jnp.float32)
    acc += jnp.dot(pool, w2_ref[...], preferred_element_type=jnp.float32)
    o_ref[...] = jax.nn.relu(acc)


def _level(cur, p, static, W1, W2):
    return pl.pallas_call(
        _level_body,
        grid=(N // _BR,),
        in_specs=[
            pl.BlockSpec((_BR, D), lambda i: (i, 0)),
            pl.BlockSpec((NC, _BR, D), lambda i: (0, i, 0)),
            pl.BlockSpec((_BR, D), lambda i: (i, 0)),
            pl.BlockSpec((D, D), lambda i: (0, 0)),
            pl.BlockSpec((D, D), lambda i: (0, 0)),
        ],
        out_specs=pl.BlockSpec((_BR, D), lambda i: (i, 0)),
        out_shape=jax.ShapeDtypeStruct((N, D), jnp.float32),
    )(cur, p, static, W1, W2)


def kernel(edge_feat, edge_index, W_e2l, W0, W11, W21, W12, W22, W13, W23):
    src = edge_index[0]
    dst = edge_index[1]
    zeros = jnp.zeros((CH, D), jnp.float32)

    edge_lin = _edge_linear(edge_feat, W_e2l)
    p = _segsum_linear(edge_lin, src, dst, zeros)
    static, cur = _combine0(p, W0)
    for W1, W2 in ((W11, W21), (W12, W22), (W13, W23)):
        p = _segsum_gather(cur, src, dst, zeros)
        cur = _level(cur, p, static, W1, W2)
    return cur
